# baseline (speedup 1.0000x reference)
import functools

import jax
import jax.numpy as jnp
from jax import lax
from jax.experimental import pallas as pl
from jax.experimental.pallas import tpu as pltpu
from jax.experimental.pallas import tpu_sc as plsc

B, L, D = 8, 1024, 32
K = 8192
NUM_Q = 8
N = B * L

T = 512
NB = 8
KB = K // NB
CDIM = NB * D

CW = 512

NW = 32
BPW = N // NW


def _stage_kernel(r_ref, qp_ref, qacc_ref, cbt_ref, c2_ref,
                  idx_ref, rout_ref, qaccout_ref, bd_ref):
    @pl.when(pl.program_id(0) == 0)
    def _():
        bd_ref[...] = jnp.zeros((CDIM, K), jnp.float32)
        for b in range(NB):
            bd_ref[D * b:D * (b + 1), KB * b:KB * (b + 1)] = \
                cbt_ref[:, KB * b:KB * (b + 1)]

    qp = qp_ref[...]
    r = r_ref[...] - qp
    rout_ref[...] = r
    qaccout_ref[...] = qacc_ref[...] + qp
    a2 = jnp.sum(r * r, axis=1, keepdims=True)
    r2 = r + r
    rrep = jnp.concatenate([r2] * NB, axis=1)
    dot2 = lax.dot_general(rrep, bd_ref[...], (((1,), (0,)), ((), ())),
                           preferred_element_type=jnp.float32)
    m = jnp.full((T, 1), jnp.inf, dtype=jnp.float32)
    am = jnp.zeros((T, 1), dtype=jnp.int32)
    for c in range(K // CW):
        sl = slice(c * CW, (c + 1) * CW)
        d2 = (a2 + c2_ref[:, sl]) - dot2[:, sl]
        dist = jnp.sqrt(jnp.maximum(d2, 0.0))
        cm = jnp.min(dist, axis=1, keepdims=True)
        ids = lax.broadcasted_iota(jnp.int32, (T, CW), 1) + c * CW
        ci = jnp.min(jnp.where(dist == cm, ids, K), axis=1, keepdims=True)
        upd = cm < m
        m = jnp.where(upd, cm, m)
        am = jnp.where(upd, ci, am)
    idx_ref[...] = am


def _tc_stage(r, qp, qacc, cbt_q, c2_q):
    return pl.pallas_call(
        _stage_kernel,
        grid=(N // T,),
        in_specs=[
            pl.BlockSpec((T, D), lambda i: (i, 0)),
            pl.BlockSpec((T, D), lambda i: (i, 0)),
            pl.BlockSpec((T, D), lambda i: (i, 0)),
            pl.BlockSpec((D, K), lambda i: (0, 0)),
            pl.BlockSpec((1, K), lambda i: (0, 0)),
        ],
        out_specs=[
            pl.BlockSpec((T, 1), lambda i: (i, 0)),
            pl.BlockSpec((T, D), lambda i: (i, 0)),
            pl.BlockSpec((T, D), lambda i: (i, 0)),
        ],
        out_shape=[
            jax.ShapeDtypeStruct((N, 1), jnp.int32),
            jax.ShapeDtypeStruct((N, D), jnp.float32),
            jax.ShapeDtypeStruct((N, D), jnp.float32),
        ],
        scratch_shapes=[pltpu.VMEM((CDIM, K), jnp.float32)],
    )(r, qp, qacc, cbt_q, c2_q)


_SC_MESH = plsc.VectorSubcoreMesh(core_axis_name="c", subcore_axis_name="s")
DPAD = 128


@functools.partial(
    pl.kernel, mesh=_SC_MESH,
    out_type=jax.ShapeDtypeStruct((N, DPAD), jnp.float32),
    scratch_types=[
        pltpu.VMEM((BPW,), jnp.int32),
        pltpu.VMEM((BPW, DPAD), jnp.float32),
        pltpu.SemaphoreType.DMA,
    ],
)
def _sc_gather(table_hbm, idx_hbm, out_hbm, idx_v, rows_v, sem):
    wid = lax.axis_index("s") * 2 + lax.axis_index("c")
    base = wid * BPW
    pltpu.sync_copy(idx_hbm.at[pl.ds(base, BPW)], idx_v)
    pltpu.async_copy(table_hbm.at[idx_v], rows_v, sem).wait()
    pltpu.sync_copy(rows_v, out_hbm.at[pl.ds(base, BPW)])


def _final_add_kernel(a_ref, b_ref, o_ref):
    o_ref[...] = a_ref[...] + b_ref[...]


def kernel(x, codebooks):
    xf = x.reshape(N, D)
    cbt = codebooks.transpose(0, 2, 1)
    c2 = jnp.sum(codebooks * codebooks, axis=-1)[:, None, :]
    cb_pad = jnp.pad(codebooks, ((0, 0), (0, 0), (0, DPAD - D)))
    zeros = jnp.zeros((N, D), jnp.float32)

    r, qp, qacc = xf, zeros, zeros
    idx_cols = []
    for q in range(NUM_Q):
        idx_q, r, qacc = _tc_stage(r, qp, qacc, cbt[q], c2[q])
        qp = _sc_gather(cb_pad[q], idx_q.reshape(N))[:, :D]
        idx_cols.append(idx_q)

    quantized = pl.pallas_call(
        _final_add_kernel,
        out_shape=jax.ShapeDtypeStruct((N, D), jnp.float32),
    )(qacc, qp)

    indices = jnp.concatenate(idx_cols, axis=1)
    indices = indices.reshape(B, L, NUM_Q).transpose(0, 2, 1)
    return (indices, quantized.reshape(B, L, D))


# speedup vs baseline: 1.0786x; 1.0786x over previous
import functools

import jax
import jax.numpy as jnp
from jax import lax
from jax.experimental import pallas as pl
from jax.experimental.pallas import tpu as pltpu
from jax.experimental.pallas import tpu_sc as plsc

B, L, D = 8, 1024, 32
K = 8192
NUM_Q = 8
N = B * L

T = 512
NB = 8
KB = K // NB
CDIM = NB * D

CW = 512

NW = 32
BPW = N // NW


def _stage_kernel(r_ref, qp_ref, qacc_ref, cbt_ref, c2_ref,
                  idx_ref, rout_ref, qaccout_ref, bd_ref):
    @pl.when(pl.program_id(0) == 0)
    def _():
        bd_ref[...] = jnp.zeros((CDIM, K), jnp.float32)
        for b in range(NB):
            bd_ref[D * b:D * (b + 1), KB * b:KB * (b + 1)] = \
                cbt_ref[:, KB * b:KB * (b + 1)]

    qp = qp_ref[...]
    r = r_ref[...] - qp
    rout_ref[...] = r
    qaccout_ref[...] = qacc_ref[...] + qp
    a2 = jnp.sum(r * r, axis=1, keepdims=True)
    r2 = r + r
    a2t = a2.T
    rrep = jnp.concatenate([r2.T] * NB, axis=0)
    dot2 = lax.dot_general(bd_ref[...], rrep, (((0,), (0,)), ((), ())),
                           preferred_element_type=jnp.float32)
    d2 = (a2t + c2_ref[...]) - dot2
    dist = jnp.sqrt(jnp.maximum(d2, 0.0))
    m = jnp.min(dist, axis=0, keepdims=True)
    ids = lax.broadcasted_iota(jnp.int32, (K, T), 0)
    am = jnp.min(jnp.where(dist == m, ids, K), axis=0, keepdims=True)
    idx_ref[...] = am


def _tc_stage(r, qp, qacc, cbt_q, c2_q):
    return pl.pallas_call(
        _stage_kernel,
        grid=(N // T,),
        in_specs=[
            pl.BlockSpec((T, D), lambda i: (i, 0)),
            pl.BlockSpec((T, D), lambda i: (i, 0)),
            pl.BlockSpec((T, D), lambda i: (i, 0)),
            pl.BlockSpec((D, K), lambda i: (0, 0)),
            pl.BlockSpec((K, 1), lambda i: (0, 0)),
        ],
        out_specs=[
            pl.BlockSpec((1, T), lambda i: (0, i)),
            pl.BlockSpec((T, D), lambda i: (i, 0)),
            pl.BlockSpec((T, D), lambda i: (i, 0)),
        ],
        out_shape=[
            jax.ShapeDtypeStruct((1, N), jnp.int32),
            jax.ShapeDtypeStruct((N, D), jnp.float32),
            jax.ShapeDtypeStruct((N, D), jnp.float32),
        ],
        scratch_shapes=[pltpu.VMEM((CDIM, K), jnp.float32)],
    )(r, qp, qacc, cbt_q, c2_q)


_SC_MESH = plsc.VectorSubcoreMesh(core_axis_name="c", subcore_axis_name="s")
DPAD = 128


@functools.partial(
    pl.kernel, mesh=_SC_MESH,
    out_type=jax.ShapeDtypeStruct((N, DPAD), jnp.float32),
    scratch_types=[
        pltpu.VMEM((BPW,), jnp.int32),
        pltpu.VMEM((BPW, DPAD), jnp.float32),
        pltpu.SemaphoreType.DMA,
    ],
)
def _sc_gather(table_hbm, idx_hbm, out_hbm, idx_v, rows_v, sem):
    wid = lax.axis_index("s") * 2 + lax.axis_index("c")
    base = wid * BPW
    pltpu.sync_copy(idx_hbm.at[pl.ds(base, BPW)], idx_v)
    pltpu.async_copy(table_hbm.at[idx_v], rows_v, sem).wait()
    pltpu.sync_copy(rows_v, out_hbm.at[pl.ds(base, BPW)])


def _final_add_kernel(a_ref, b_ref, o_ref):
    o_ref[...] = a_ref[...] + b_ref[...]


def kernel(x, codebooks):
    xf = x.reshape(N, D)
    cbt = codebooks.transpose(0, 2, 1)
    c2 = jnp.sum(codebooks * codebooks, axis=-1)[:, :, None]
    cb_pad = jnp.pad(codebooks, ((0, 0), (0, 0), (0, DPAD - D)))
    zeros = jnp.zeros((N, D), jnp.float32)

    r, qp, qacc = xf, zeros, zeros
    idx_cols = []
    for q in range(NUM_Q):
        idx_q, r, qacc = _tc_stage(r, qp, qacc, cbt[q], c2[q])
        qp = _sc_gather(cb_pad[q], idx_q.reshape(N))[:, :D]
        idx_cols.append(idx_q)

    quantized = pl.pallas_call(
        _final_add_kernel,
        out_shape=jax.ShapeDtypeStruct((N, D), jnp.float32),
    )(qacc, qp)

    indices = jnp.concatenate(idx_cols, axis=0)
    indices = indices.reshape(NUM_Q, B, L).transpose(1, 0, 2)
    return (indices, quantized.reshape(B, L, D))


# speedup vs baseline: 1.4082x; 1.3057x over previous
import functools

import jax
import jax.numpy as jnp
from jax import lax
from jax.experimental import pallas as pl
from jax.experimental.pallas import tpu as pltpu
from jax.experimental.pallas import tpu_sc as plsc

B, L, D = 8, 1024, 32
K = 8192
NUM_Q = 8
N = B * L

T = 512
NB = 8
KB = K // NB
CDIM = NB * D

CW = 512

NW = 32
BPW = N // NW


def _stage_kernel(r_ref, qp_ref, qacc_ref, cbt_ref, c2_ref,
                  idx_ref, rout_ref, qaccout_ref, bd_ref):
    @pl.when(pl.program_id(0) == 0)
    def _():
        bd_ref[...] = jnp.zeros((CDIM, K), jnp.float32)
        for b in range(NB):
            bd_ref[D * b:D * (b + 1), KB * b:KB * (b + 1)] = \
                cbt_ref[:, KB * b:KB * (b + 1)]

    qp = qp_ref[...]
    r = r_ref[...] - qp
    rout_ref[...] = r
    qaccout_ref[...] = qacc_ref[...] + qp
    a2 = jnp.sum(r * r, axis=1, keepdims=True)
    r2 = r + r
    rrep = jnp.concatenate([r2] * NB, axis=1)
    dot2 = lax.dot_general(rrep, bd_ref[...], (((1,), (0,)), ((), ())),
                           preferred_element_type=jnp.float32)
    d2 = (a2 + c2_ref[...]) - dot2
    d2 = jnp.maximum(d2, 0.0)
    dist = jnp.where(d2 == 0.0, 0.0, d2 * lax.rsqrt(d2))
    am = jnp.argmin(dist, axis=1, keepdims=True).astype(jnp.int32)
    idx_ref[...] = am


def _tc_stage(r, qp, qacc, cbt_q, c2_q):
    return pl.pallas_call(
        _stage_kernel,
        grid=(N // T,),
        in_specs=[
            pl.BlockSpec((T, D), lambda i: (i, 0)),
            pl.BlockSpec((T, D), lambda i: (i, 0)),
            pl.BlockSpec((T, D), lambda i: (i, 0)),
            pl.BlockSpec((D, K), lambda i: (0, 0)),
            pl.BlockSpec((1, K), lambda i: (0, 0)),
        ],
        out_specs=[
            pl.BlockSpec((T, 1), lambda i: (i, 0)),
            pl.BlockSpec((T, D), lambda i: (i, 0)),
            pl.BlockSpec((T, D), lambda i: (i, 0)),
        ],
        out_shape=[
            jax.ShapeDtypeStruct((N, 1), jnp.int32),
            jax.ShapeDtypeStruct((N, D), jnp.float32),
            jax.ShapeDtypeStruct((N, D), jnp.float32),
        ],
        scratch_shapes=[pltpu.VMEM((CDIM, K), jnp.float32)],
    )(r, qp, qacc, cbt_q, c2_q)


_SC_MESH = plsc.VectorSubcoreMesh(core_axis_name="c", subcore_axis_name="s")
DPAD = 128


@functools.partial(
    pl.kernel, mesh=_SC_MESH,
    out_type=jax.ShapeDtypeStruct((N, DPAD), jnp.float32),
    scratch_types=[
        pltpu.VMEM((BPW,), jnp.int32),
        pltpu.VMEM((BPW, DPAD), jnp.float32),
        pltpu.SemaphoreType.DMA,
    ],
)
def _sc_gather(table_hbm, idx_hbm, out_hbm, idx_v, rows_v, sem):
    wid = lax.axis_index("s") * 2 + lax.axis_index("c")
    base = wid * BPW
    pltpu.sync_copy(idx_hbm.at[pl.ds(base, BPW)], idx_v)
    pltpu.async_copy(table_hbm.at[idx_v], rows_v, sem).wait()
    pltpu.sync_copy(rows_v, out_hbm.at[pl.ds(base, BPW)])


def _final_add_kernel(a_ref, b_ref, o_ref):
    o_ref[...] = a_ref[...] + b_ref[...]


def kernel(x, codebooks):
    xf = x.reshape(N, D)
    cbt = codebooks.transpose(0, 2, 1)
    c2 = jnp.sum(codebooks * codebooks, axis=-1)[:, None, :]
    cb_pad = jnp.pad(codebooks, ((0, 0), (0, 0), (0, DPAD - D)))
    zeros = jnp.zeros((N, D), jnp.float32)

    r, qp, qacc = xf, zeros, zeros
    idx_cols = []
    for q in range(NUM_Q):
        idx_q, r, qacc = _tc_stage(r, qp, qacc, cbt[q], c2[q])
        qp = _sc_gather(cb_pad[q], idx_q.reshape(N))[:, :D]
        idx_cols.append(idx_q)

    quantized = pl.pallas_call(
        _final_add_kernel,
        out_shape=jax.ShapeDtypeStruct((N, D), jnp.float32),
    )(qacc, qp)

    indices = jnp.concatenate(idx_cols, axis=1)
    indices = indices.reshape(B, L, NUM_Q).transpose(0, 2, 1)
    return (indices, quantized.reshape(B, L, D))
